# baseline (speedup 1.0000x reference)
import jax
import jax.numpy as jnp
from jax.experimental import pallas as pl
from jax.experimental.pallas import tpu as pltpu

_NUM_CLASSES = 80
_IGNORE_THRESH = 0.5


def _body(z_ref, t_ref, anc_ref, out_ref):
    t = t_ref[...]
    B = t.shape[0]
    T = t.shape[1]
    G = z_ref.shape[2]
    A = z_ref.shape[0] // B

    tsum = (t[:, :, 0:1] + t[:, :, 1:2] + t[:, :, 2:3]
            + t[:, :, 3:4] + t[:, :, 4:5])
    valid = tsum != 0.0
    gx = t[:, :, 1:2] * G
    gy = t[:, :, 2:3] * G
    gw = t[:, :, 3:4] * G
    gh = t[:, :, 4:5] * G
    gi = gx.astype(jnp.int32)
    gj = gy.astype(jnp.int32)

    ious = []
    for a in range(A):
        aw = anc_ref[a, 0]
        ah = anc_ref[a, 1]
        iw = jnp.clip(jnp.minimum(gw / 2, aw / 2) - jnp.maximum(-gw / 2, -aw / 2) + 1.0, 0.0, None)
        ih = jnp.clip(jnp.minimum(gh / 2, ah / 2) - jnp.maximum(-gh / 2, -ah / 2) + 1.0, 0.0, None)
        inter = iw * ih
        a1 = (gw + 1.0) * (gh + 1.0)
        a2 = (aw + 1.0) * (ah + 1.0)
        ious.append(inter / (a1 + a2 - inter + 1e-16))
    i0, i1, i2 = ious
    b01 = i1 > i0
    best_iou = jnp.where(b01, i1, i0)
    best_n = jnp.where(b01, 1, 0)
    b2 = i2 > best_iou
    best_iou = jnp.where(b2, i2, best_iou)
    best_n = jnp.where(b2, 2, best_n)
    cond_ign = valid & (best_iou > _IGNORE_THRESH)

    hi = jnp.where(cond_ign, best_n * G + gj, A * G)
    row_iota = jax.lax.broadcasted_iota(jnp.int32, (B, T, A * G), 2)
    u = jnp.where(hi == row_iota, 1.0, 0.0)
    col_iota = jax.lax.broadcasted_iota(jnp.int32, (B, T, G), 2)
    v = jnp.where(gi == col_iota, 1.0, 0.0)

    count = jax.lax.dot_general(
        u, v,
        dimension_numbers=(((1,), (1,)), ((0,), (0,))),
        preferred_element_type=jnp.float32,
    )

    total = jnp.float32(0.0)
    for a in range(A):
        z = z_ref[a * B:(a + 1) * B]
        s = jax.nn.sigmoid(z)
        f = -jnp.maximum(jnp.log(1.0 - s), -100.0)
        keep = count[:, a * G:(a + 1) * G, :] < 0.5
        total = total + jnp.sum(jnp.where(keep, f, 0.0))
    out_ref[0, 0] = total


def kernel(pred, target, anchors, num_anchors, grid_size):
    B, C, G, _ = pred.shape
    A = anchors.shape[0]
    attrs = C // A
    scaled_anchors = (anchors / (grid_size // G)) * (num_anchors // A)
    resh = pred.reshape(B * C, G, G)
    idx = jnp.array([b * C + a * attrs + 4 for a in range(A) for b in range(B)],
                    dtype=jnp.int32)
    conf = jnp.take(resh, idx, axis=0)

    out = pl.pallas_call(
        _body,
        grid=(1,),
        out_shape=jax.ShapeDtypeStruct((1, 1), jnp.float32),
        in_specs=[
            pl.BlockSpec(conf.shape, lambda i: (0, 0, 0)),
            pl.BlockSpec(target.shape, lambda i: (0, 0, 0)),
            pl.BlockSpec(memory_space=pltpu.SMEM),
        ],
        out_specs=pl.BlockSpec(memory_space=pltpu.SMEM),
    )(conf, target, scaled_anchors)
    return out[0, 0]


# speedup vs baseline: 1.9594x; 1.9594x over previous
import jax
import jax.numpy as jnp
from jax.experimental import pallas as pl
from jax.experimental.pallas import tpu as pltpu

_NUM_CLASSES = 80
_IGNORE_THRESH = 0.5


def _make_extract(A, attrs):
    def _extract(tp_ref, o0_ref, o1_ref, o2_ref):
        x = tp_ref[...]
        outs = (o0_ref, o1_ref, o2_ref)
        for a in range(A):
            c = a * attrs + 4
            outs[a][...] = x[:, :, :, c:c + 1]
    return _extract


def _make_loss(B, T, G, A):
    N = G * G

    def _loss(z0_ref, z1_ref, z2_ref, t_ref, anc_ref, out_ref):
        t = t_ref[...]
        tsum = (t[:, :, 0:1] + t[:, :, 1:2] + t[:, :, 2:3]
                + t[:, :, 3:4] + t[:, :, 4:5])
        valid = tsum != 0.0
        gx = t[:, :, 1:2] * G
        gy = t[:, :, 2:3] * G
        gw = t[:, :, 3:4] * G
        gh = t[:, :, 4:5] * G
        gi = gx.astype(jnp.int32)
        gj = gy.astype(jnp.int32)

        ious = []
        for a in range(A):
            aw = anc_ref[a, 0]
            ah = anc_ref[a, 1]
            iw = jnp.clip(jnp.minimum(gw / 2, aw / 2) - jnp.maximum(-gw / 2, -aw / 2) + 1.0, 0.0, None)
            ih = jnp.clip(jnp.minimum(gh / 2, ah / 2) - jnp.maximum(-gh / 2, -ah / 2) + 1.0, 0.0, None)
            inter = iw * ih
            a1 = (gw + 1.0) * (gh + 1.0)
            a2 = (aw + 1.0) * (ah + 1.0)
            ious.append(inter / (a1 + a2 - inter + 1e-16))
        i0, i1, i2 = ious
        b01 = i1 > i0
        best_iou = jnp.where(b01, i1, i0)
        best_n = jnp.where(b01, 1, 0)
        b2 = i2 > best_iou
        best_iou = jnp.where(b2, i2, best_iou)
        best_n = jnp.where(b2, 2, best_n)
        cond_ign = valid & (best_iou > _IGNORE_THRESH)
        cell = gj * G + gi

        ones = jnp.ones((B, T), jnp.float32)
        col_iota = jax.lax.broadcasted_iota(jnp.int32, (B, T, N), 2)
        total = jnp.float32(0.0)
        for a, z_ref in enumerate((z0_ref, z1_ref, z2_ref)):
            key = jnp.where(cond_ign & (best_n == a), cell, -1)
            onehot = jnp.where(key == col_iota, 1.0, 0.0)
            count = jax.lax.dot_general(
                ones, onehot,
                dimension_numbers=(((1,), (1,)), ((0,), (0,))),
                preferred_element_type=jnp.float32,
            )
            z = z_ref[...]
            s = jax.nn.sigmoid(z)
            f = -jnp.maximum(jnp.log(1.0 - s), -100.0)
            total = total + jnp.sum(jnp.where(count < 0.5, f, 0.0))
        out_ref[0, 0] = total
    return _loss


def kernel(pred, target, anchors, num_anchors, grid_size):
    B, C, G, _ = pred.shape
    A = anchors.shape[0]
    T = target.shape[1]
    attrs = C // A
    scaled_anchors = (anchors / (grid_size // G)) * (num_anchors // A)

    tp = jnp.transpose(pred, (2, 3, 0, 1))

    zouts = pl.pallas_call(
        _make_extract(A, attrs),
        grid=(G,),
        out_shape=[jax.ShapeDtypeStruct((G, G, B, 1), jnp.float32)] * A,
        in_specs=[pl.BlockSpec((1, G, B, C), lambda j: (j, 0, 0, 0))],
        out_specs=[pl.BlockSpec((1, G, B, 1), lambda j: (j, 0, 0, 0))] * A,
    )(tp)

    zs = [z.reshape(G * G, B).T for z in zouts]

    out = pl.pallas_call(
        _make_loss(B, T, G, A),
        grid=(1,),
        out_shape=jax.ShapeDtypeStruct((1, 1), jnp.float32),
        in_specs=[
            pl.BlockSpec((B, G * G), lambda i: (0, 0)),
            pl.BlockSpec((B, G * G), lambda i: (0, 0)),
            pl.BlockSpec((B, G * G), lambda i: (0, 0)),
            pl.BlockSpec(target.shape, lambda i: (0, 0, 0)),
            pl.BlockSpec(memory_space=pltpu.SMEM),
        ],
        out_specs=pl.BlockSpec(memory_space=pltpu.SMEM),
    )(*zs, target, scaled_anchors)
    return out[0, 0]


# speedup vs baseline: 5.3871x; 2.7493x over previous
import jax
import jax.numpy as jnp
from jax.experimental import pallas as pl
from jax.experimental.pallas import tpu as pltpu

_NUM_CLASSES = 80
_IGNORE_THRESH = 0.5


def _make_body(B, T, G, A, attrs):
    NB = T * B

    def _body(tp_ref, tl_ref, anc_ref, out_ref, z0_scr, z1_scr, z2_scr):
        j = pl.program_id(0)
        x = tp_ref[...][0]
        scrs = (z0_scr, z1_scr, z2_scr)
        for a in range(A):
            c = a * attrs + 4
            za = x[:, :, c:c + 1].reshape(G, B)
            scrs[a][pl.ds(j * B, B), :] = za.T

        @pl.when(j == G - 1)
        def _finish():
            t = tl_ref[...]
            t0, t1, t2, t3, t4 = t[0], t[1], t[2], t[3], t[4]
            valid = (t0 + t1 + t2 + t3 + t4) != 0.0
            gx = t1 * G
            gy = t2 * G
            gw = t3 * G
            gh = t4 * G
            gi = gx.astype(jnp.int32)
            gj = gy.astype(jnp.int32)

            ious = []
            for a in range(A):
                aw = anc_ref[a, 0]
                ah = anc_ref[a, 1]
                iw = jnp.clip(jnp.minimum(gw / 2, aw / 2) - jnp.maximum(-gw / 2, -aw / 2) + 1.0, 0.0, None)
                ih = jnp.clip(jnp.minimum(gh / 2, ah / 2) - jnp.maximum(-gh / 2, -ah / 2) + 1.0, 0.0, None)
                inter = iw * ih
                a1 = (gw + 1.0) * (gh + 1.0)
                a2 = (aw + 1.0) * (ah + 1.0)
                ious.append(inter / (a1 + a2 - inter + 1e-16))
            i0, i1, i2 = ious
            b01 = i1 > i0
            best_iou = jnp.where(b01, i1, i0)
            best_n = jnp.where(b01, 1, 0)
            b2 = i2 > best_iou
            best_iou = jnp.where(b2, i2, best_iou)
            best_n = jnp.where(b2, 2, best_n)
            cond_ign = valid & (best_iou > _IGNORE_THRESH)

            b_idx = jax.lax.broadcasted_iota(jnp.int32, (1, NB), 1) // T
            rkey = gj * B + b_idx

            row_iota = jax.lax.broadcasted_iota(jnp.int32, (G * B, NB), 0)
            col_iota = jax.lax.broadcasted_iota(jnp.int32, (G, NB), 0)
            u2 = jnp.where(gi == col_iota, 1.0, 0.0)

            total = jnp.float32(0.0)
            for a in range(A):
                key_a = jnp.where(cond_ign & (best_n == a), rkey, -1)
                u1 = jnp.where(key_a == row_iota, 1.0, 0.0)
                count = jax.lax.dot_general(
                    u1, u2,
                    dimension_numbers=(((1,), (1,)), ((), ())),
                    preferred_element_type=jnp.float32,
                )
                z = scrs[a][...]
                s = jax.nn.sigmoid(z)
                f = -jnp.maximum(jnp.log(1.0 - s), -100.0)
                total = total + jnp.sum(jnp.where(count < 0.5, f, 0.0))
            out_ref[0, 0] = total
    return _body


def kernel(pred, target, anchors, num_anchors, grid_size):
    B, C, G, _ = pred.shape
    A = anchors.shape[0]
    T = target.shape[1]
    attrs = C // A
    scaled_anchors = (anchors / (grid_size // G)) * (num_anchors // A)

    tp = jnp.transpose(pred, (2, 3, 0, 1))
    tl = jnp.transpose(target, (2, 0, 1)).reshape(5, 1, B * T)

    out = pl.pallas_call(
        _make_body(B, T, G, A, attrs),
        grid=(G,),
        out_shape=jax.ShapeDtypeStruct((1, 1), jnp.float32),
        in_specs=[
            pl.BlockSpec((1, G, B, C), lambda j: (j, 0, 0, 0)),
            pl.BlockSpec(tl.shape, lambda j: (0, 0, 0)),
            pl.BlockSpec(memory_space=pltpu.SMEM),
        ],
        out_specs=pl.BlockSpec(memory_space=pltpu.SMEM),
        scratch_shapes=[pltpu.VMEM((G * B, G), jnp.float32)] * A,
    )(tp, tl, scaled_anchors)
    return out[0, 0]


# speedup vs baseline: 6.3239x; 1.1739x over previous
import jax
import jax.numpy as jnp
from jax.experimental import pallas as pl
from jax.experimental.pallas import tpu as pltpu

_NUM_CLASSES = 80
_IGNORE_THRESH = 0.5
_ROWS = 2


def _make_body(B, T, G, A, attrs, R):
    NB = T * B

    def _body(tp_ref, tl_ref, anc_ref, out_ref,
              z0_scr, z1_scr, z2_scr, c0_scr, c1_scr, c2_scr):
        j = pl.program_id(0)
        x = tp_ref[...]
        scrs = (z0_scr, z1_scr, z2_scr)
        cnts = (c0_scr, c1_scr, c2_scr)
        for r in range(R):
            xr = x[r]
            for a in range(A):
                c = a * attrs + 4
                za = xr[:, :, c:c + 1].reshape(G, B)
                row = (j * R + r) * B
                scrs[a][pl.ds(row, B), :] = za.T

        @pl.when(j == 0)
        def _mask():
            t = tl_ref[...]
            t0, t1, t2, t3, t4 = t[0], t[1], t[2], t[3], t[4]
            valid = (t0 + t1 + t2 + t3 + t4) != 0.0
            gx = t1 * G
            gy = t2 * G
            gw = t3 * G
            gh = t4 * G
            gi = gx.astype(jnp.int32)
            gj = gy.astype(jnp.int32)

            ious = []
            for a in range(A):
                aw = anc_ref[a, 0]
                ah = anc_ref[a, 1]
                iw = jnp.clip(jnp.minimum(gw / 2, aw / 2) - jnp.maximum(-gw / 2, -aw / 2) + 1.0, 0.0, None)
                ih = jnp.clip(jnp.minimum(gh / 2, ah / 2) - jnp.maximum(-gh / 2, -ah / 2) + 1.0, 0.0, None)
                inter = iw * ih
                a1 = (gw + 1.0) * (gh + 1.0)
                a2 = (aw + 1.0) * (ah + 1.0)
                ious.append(inter / (a1 + a2 - inter + 1e-16))
            i0, i1, i2 = ious
            b01 = i1 > i0
            best_iou = jnp.where(b01, i1, i0)
            best_n = jnp.where(b01, 1, 0)
            b2 = i2 > best_iou
            best_iou = jnp.where(b2, i2, best_iou)
            best_n = jnp.where(b2, 2, best_n)
            cond_ign = valid & (best_iou > _IGNORE_THRESH)

            b_idx = jax.lax.broadcasted_iota(jnp.int32, (1, NB), 1) // T
            rkey = gj * B + b_idx

            row_iota = jax.lax.broadcasted_iota(jnp.int32, (G * B, NB), 0)
            col_iota = jax.lax.broadcasted_iota(jnp.int32, (G, NB), 0)
            u2 = jnp.where(gi == col_iota, 1.0, 0.0)

            for a in range(A):
                key_a = jnp.where(cond_ign & (best_n == a), rkey, -1)
                u1 = jnp.where(key_a == row_iota, 1.0, 0.0)
                cnts[a][...] = jax.lax.dot_general(
                    u1, u2,
                    dimension_numbers=(((1,), (1,)), ((), ())),
                    preferred_element_type=jnp.float32,
                )

        @pl.when(j == G // R - 1)
        def _finish():
            total = jnp.float32(0.0)
            for a in range(A):
                z = scrs[a][...]
                s = jax.nn.sigmoid(z)
                f = -jnp.maximum(jnp.log(1.0 - s), -100.0)
                total = total + jnp.sum(jnp.where(cnts[a][...] < 0.5, f, 0.0))
            out_ref[0, 0] = total
    return _body


def kernel(pred, target, anchors, num_anchors, grid_size):
    B, C, G, _ = pred.shape
    A = anchors.shape[0]
    T = target.shape[1]
    attrs = C // A
    R = _ROWS if G % _ROWS == 0 else 1
    scaled_anchors = (anchors / (grid_size // G)) * (num_anchors // A)

    tp = jnp.transpose(pred, (2, 3, 0, 1))
    tl = jnp.transpose(target, (2, 0, 1)).reshape(5, 1, B * T)

    out = pl.pallas_call(
        _make_body(B, T, G, A, attrs, R),
        grid=(G // R,),
        out_shape=jax.ShapeDtypeStruct((1, 1), jnp.float32),
        in_specs=[
            pl.BlockSpec((R, G, B, C), lambda j: (j, 0, 0, 0)),
            pl.BlockSpec(tl.shape, lambda j: (0, 0, 0)),
            pl.BlockSpec(memory_space=pltpu.SMEM),
        ],
        out_specs=pl.BlockSpec(memory_space=pltpu.SMEM),
        scratch_shapes=[pltpu.VMEM((G * B, G), jnp.float32)] * (2 * A),
    )(tp, tl, scaled_anchors)
    return out[0, 0]


# speedup vs baseline: 7.2622x; 1.1484x over previous
import jax
import jax.numpy as jnp
from jax.experimental import pallas as pl
from jax.experimental.pallas import tpu as pltpu

_NUM_CLASSES = 80
_IGNORE_THRESH = 0.5
_ROWS = 4


def _make_body(B, T, G, A, attrs, R):
    NB = T * B

    def _body(tp_ref, tl_ref, anc_ref, out_ref,
              z0_scr, z1_scr, z2_scr, c0_scr, c1_scr, c2_scr):
        j = pl.program_id(0)
        x = tp_ref[...]
        scrs = (z0_scr, z1_scr, z2_scr)
        cnts = (c0_scr, c1_scr, c2_scr)
        for r in range(R):
            xr = x[r]
            for a in range(A):
                c = a * attrs + 4
                za = xr[:, :, c:c + 1].reshape(G, B)
                row = (j * R + r) * B
                scrs[a][pl.ds(row, B), :] = za.T

        @pl.when(j == 0)
        def _mask():
            t = tl_ref[...]
            t0, t1, t2, t3, t4 = t[0], t[1], t[2], t[3], t[4]
            valid = (t0 + t1 + t2 + t3 + t4) != 0.0
            gx = t1 * G
            gy = t2 * G
            gw = t3 * G
            gh = t4 * G
            gi = gx.astype(jnp.int32)
            gj = gy.astype(jnp.int32)

            ious = []
            for a in range(A):
                aw = anc_ref[a, 0]
                ah = anc_ref[a, 1]
                iw = jnp.clip(jnp.minimum(gw / 2, aw / 2) - jnp.maximum(-gw / 2, -aw / 2) + 1.0, 0.0, None)
                ih = jnp.clip(jnp.minimum(gh / 2, ah / 2) - jnp.maximum(-gh / 2, -ah / 2) + 1.0, 0.0, None)
                inter = iw * ih
                a1 = (gw + 1.0) * (gh + 1.0)
                a2 = (aw + 1.0) * (ah + 1.0)
                ious.append(inter / (a1 + a2 - inter + 1e-16))
            i0, i1, i2 = ious
            b01 = i1 > i0
            best_iou = jnp.where(b01, i1, i0)
            best_n = jnp.where(b01, 1, 0)
            b2 = i2 > best_iou
            best_iou = jnp.where(b2, i2, best_iou)
            best_n = jnp.where(b2, 2, best_n)
            cond_ign = valid & (best_iou > _IGNORE_THRESH)

            b_idx = jax.lax.broadcasted_iota(jnp.int32, (1, NB), 1) // T
            rkey = gj * B + b_idx

            row_iota = jax.lax.broadcasted_iota(jnp.int32, (G * B, NB), 0)
            col_iota = jax.lax.broadcasted_iota(jnp.int32, (G, NB), 0)
            u2 = jnp.where(gi == col_iota, 1.0, 0.0)

            for a in range(A):
                key_a = jnp.where(cond_ign & (best_n == a), rkey, -1)
                u1 = jnp.where(key_a == row_iota, 1.0, 0.0)
                cnts[a][...] = jax.lax.dot_general(
                    u1, u2,
                    dimension_numbers=(((1,), (1,)), ((), ())),
                    preferred_element_type=jnp.float32,
                )

        @pl.when(j == G // R - 1)
        def _finish():
            total = jnp.float32(0.0)
            for a in range(A):
                z = scrs[a][...]
                s = jax.nn.sigmoid(z)
                f = -jnp.maximum(jnp.log(1.0 - s), -100.0)
                total = total + jnp.sum(jnp.where(cnts[a][...] < 0.5, f, 0.0))
            out_ref[0, 0] = total
    return _body


def kernel(pred, target, anchors, num_anchors, grid_size):
    B, C, G, _ = pred.shape
    A = anchors.shape[0]
    T = target.shape[1]
    attrs = C // A
    R = _ROWS if G % _ROWS == 0 else 1
    scaled_anchors = (anchors / (grid_size // G)) * (num_anchors // A)

    tp = jnp.transpose(pred, (2, 3, 0, 1))
    tl = jnp.transpose(target, (2, 0, 1)).reshape(5, 1, B * T)

    out = pl.pallas_call(
        _make_body(B, T, G, A, attrs, R),
        grid=(G // R,),
        out_shape=jax.ShapeDtypeStruct((1, 1), jnp.float32),
        in_specs=[
            pl.BlockSpec((R, G, B, C), lambda j: (j, 0, 0, 0)),
            pl.BlockSpec(tl.shape, lambda j: (0, 0, 0)),
            pl.BlockSpec(memory_space=pltpu.SMEM),
        ],
        out_specs=pl.BlockSpec(memory_space=pltpu.SMEM),
        scratch_shapes=[pltpu.VMEM((G * B, G), jnp.float32)] * (2 * A),
    )(tp, tl, scaled_anchors)
    return out[0, 0]
